# baseline (speedup 1.0000x reference)
import functools

import jax
import jax.numpy as jnp
from jax import lax
from jax.experimental import pallas as pl
from jax.experimental.pallas import tpu as pltpu
from jax.experimental.pallas import tpu_sc as plsc

VOCAB = 90
D = 128
BATCH = 16384
HIST = 200
N = BATCH * HIST
NUM_CORES = 2
NUM_SUBCORES = 16
NW = NUM_CORES * NUM_SUBCORES
PER_W = N // NW
CS = 192
CV = 64
GROUP = CS + CV
NGROUP = PER_W // GROUP

assert PER_W * NW == N
assert NGROUP * GROUP == PER_W
assert CS % 8 == 0 and CV % 8 == 0 and PER_W % 8 == 0
assert CV % 16 == 0
assert D % 16 == 0


def _build_kernel():
  mesh = plsc.VectorSubcoreMesh(core_axis_name="c", subcore_axis_name="s")

  @functools.partial(
      pl.kernel,
      mesh=mesh,
      out_type=jax.ShapeDtypeStruct((N, D), jnp.float32),
      scratch_types=(
          [pltpu.VMEM_SHARED((VOCAB, D), jnp.float32)]
          + [pltpu.VMEM((VOCAB, D), jnp.float32)]
          + [pltpu.VMEM((GROUP,), jnp.int32) for _ in range(2)]
          + [pltpu.VMEM((CS, D), jnp.float32) for _ in range(2)]
          + [pltpu.VMEM((CV, D), jnp.float32) for _ in range(2)]
          + [pltpu.SemaphoreType.DMA for _ in range(8)]
      ),
  )
  def emb_kernel(idx_hbm, table_hbm, out_hbm, shared_tab, local_tab,
                 idx0, idx1, srow0, srow1, vrow0, vrow1,
                 gsem0, gsem1, ssem0, ssem1, vsem0, vsem1, isem0, isem1):
    idx_v = (idx0, idx1)
    srow = (srow0, srow1)
    vrow = (vrow0, vrow1)
    gsem = (gsem0, gsem1)
    ssem = (ssem0, ssem1)
    vsem = (vsem0, vsem1)
    isem = (isem0, isem1)

    sid = lax.axis_index("s")
    wid = sid * NUM_CORES + lax.axis_index("c")
    base = wid * PER_W

    @pl.when(sid == 0)
    def _stage():
      pltpu.sync_copy(table_hbm, shared_tab)

    pltpu.sync_copy(table_hbm, local_tab)
    plsc.subcore_barrier()

    def idx_src(g):
      return idx_hbm.at[pl.ds(base + g * GROUP, GROUP)]

    def sout_dst(g):
      return out_hbm.at[pl.ds(base + g * GROUP, CS)]

    def vout_dst(g):
      return out_hbm.at[pl.ds(base + g * GROUP + CS, CV)]

    def fill_vec(b):
      def row_body(q, carry):
        iv = idx_v[b][pl.ds(CS + 16 * q, 16)]
        for u in range(16):
          r = 16 * q + u
          v = iv[u]
          for k in range(D // 16):
            vrow[b][r, pl.ds(16 * k, 16)] = local_tab[v, pl.ds(16 * k, 16)]
        return carry

      lax.fori_loop(0, CV // 16, row_body, 0)

    pltpu.async_copy(idx_src(0), idx_v[0], isem[0])
    pltpu.async_copy(idx_src(1), idx_v[1], isem[1])

    def body(g2, carry):
      for b in (0, 1):
        g = 2 * g2 + b

        @pl.when(g2 >= 1)
        def _reclaim():
          pltpu.make_async_copy(srow[b], sout_dst(g), ssem[b]).wait()
          pltpu.make_async_copy(vrow[b], vout_dst(g), vsem[b]).wait()

        pltpu.make_async_copy(idx_src(g), idx_v[b], isem[b]).wait()
        pltpu.async_copy(shared_tab.at[idx_v[b].at[pl.ds(0, CS)]],
                         srow[b], gsem[b])
        fill_vec(b)
        pltpu.async_copy(vrow[b], vout_dst(g), vsem[b])
        pltpu.make_async_copy(shared_tab.at[idx_v[b].at[pl.ds(0, CS)]],
                              srow[b], gsem[b]).wait()
        nxt = jnp.minimum(g + 1, NGROUP - 1)
        pltpu.async_copy(idx_src(nxt), idx_v[1 - b], isem[1 - b])
        pltpu.async_copy(srow[b], sout_dst(g), ssem[b])
      return carry

    lax.fori_loop(0, NGROUP // 2, body, 0)

    pltpu.make_async_copy(idx_src(0), idx_v[1], isem[1]).wait()
    for b in (0, 1):
      pltpu.make_async_copy(srow[b], sout_dst(0), ssem[b]).wait()
      pltpu.make_async_copy(vrow[b], vout_dst(0), vsem[b]).wait()

  return emb_kernel


_EMB_KERNEL = _build_kernel()


@jax.jit
def kernel(note, table):
  flat = note.reshape(-1)
  out = _EMB_KERNEL(flat, table)
  return out.reshape(BATCH, HIST, D)


# speedup vs baseline: 1.2699x; 1.2699x over previous
import functools

import jax
import jax.numpy as jnp
from jax import lax
from jax.experimental import pallas as pl
from jax.experimental.pallas import tpu as pltpu
from jax.experimental.pallas import tpu_sc as plsc

VOCAB = 90
D = 128
BATCH = 16384
HIST = 200
N = BATCH * HIST
NUM_CORES = 2
NUM_SUBCORES = 16
NW = NUM_CORES * NUM_SUBCORES
PER_W = N // NW
CS = 224
CV = 32
GROUP = CS + CV
NGROUP = PER_W // GROUP

assert PER_W * NW == N
assert NGROUP * GROUP == PER_W
assert CS % 8 == 0 and CV % 8 == 0 and PER_W % 8 == 0
assert CV % 16 == 0
assert D % 16 == 0


def _build_kernel():
  mesh = plsc.VectorSubcoreMesh(core_axis_name="c", subcore_axis_name="s")

  @functools.partial(
      pl.kernel,
      mesh=mesh,
      out_type=jax.ShapeDtypeStruct((N, D), jnp.float32),
      scratch_types=(
          [pltpu.VMEM_SHARED((VOCAB, D), jnp.float32)]
          + [pltpu.VMEM((VOCAB, D), jnp.float32)]
          + [pltpu.VMEM((CS,), jnp.int32) for _ in range(2)]
          + [pltpu.VMEM((CV,), jnp.int32) for _ in range(2)]
          + [pltpu.VMEM((CS, D), jnp.float32) for _ in range(2)]
          + [pltpu.VMEM((CV, D), jnp.float32) for _ in range(2)]
          + [pltpu.SemaphoreType.DMA for _ in range(10)]
      ),
  )
  def emb_kernel(idx_hbm, table_hbm, out_hbm, shared_tab, local_tab,
                 sidx0, sidx1, vidx0, vidx1, srow0, srow1, vrow0, vrow1,
                 gsem0, gsem1, ssem0, ssem1, vsem0, vsem1,
                 isem0, isem1, jsem0, jsem1):
    sidx = (sidx0, sidx1)
    vidx = (vidx0, vidx1)
    srow = (srow0, srow1)
    vrow = (vrow0, vrow1)
    gsem = (gsem0, gsem1)
    ssem = (ssem0, ssem1)
    vsem = (vsem0, vsem1)
    isem = (isem0, isem1)
    jsem = (jsem0, jsem1)

    sid = lax.axis_index("s")
    wid = sid * NUM_CORES + lax.axis_index("c")
    base = wid * PER_W

    @pl.when(sid == 0)
    def _stage():
      pltpu.sync_copy(table_hbm, shared_tab)

    pltpu.sync_copy(table_hbm, local_tab)
    plsc.subcore_barrier()

    def sidx_src(g):
      return idx_hbm.at[pl.ds(base + g * GROUP, CS)]

    def vidx_src(g):
      return idx_hbm.at[pl.ds(base + g * GROUP + CS, CV)]

    def sout_dst(g):
      return out_hbm.at[pl.ds(base + g * GROUP, CS)]

    def vout_dst(g):
      return out_hbm.at[pl.ds(base + g * GROUP + CS, CV)]

    def fill_vec(b):
      def row_body(q, carry):
        iv = vidx[b][pl.ds(16 * q, 16)]
        for u in range(16):
          r = 16 * q + u
          v = iv[u]
          for k in range(D // 16):
            vrow[b][r, pl.ds(16 * k, 16)] = local_tab[v, pl.ds(16 * k, 16)]
        return carry

      lax.fori_loop(0, CV // 16, row_body, 0)

    pltpu.async_copy(sidx_src(0), sidx[0], isem[0])
    pltpu.async_copy(vidx_src(0), vidx[0], jsem[0])
    pltpu.async_copy(sidx_src(1), sidx[1], isem[1])
    pltpu.async_copy(vidx_src(1), vidx[1], jsem[1])

    def body(g2, carry):
      for b in (0, 1):
        g = 2 * g2 + b

        @pl.when(g2 >= 1)
        def _reclaim():
          pltpu.make_async_copy(srow[b], sout_dst(g), ssem[b]).wait()
          pltpu.make_async_copy(vrow[b], vout_dst(g), vsem[b]).wait()

        pltpu.make_async_copy(sidx_src(g), sidx[b], isem[b]).wait()
        pltpu.async_copy(shared_tab.at[sidx[b]], srow[b], gsem[b])
        pltpu.make_async_copy(vidx_src(g), vidx[b], jsem[b]).wait()
        fill_vec(b)
        pltpu.async_copy(vrow[b], vout_dst(g), vsem[b])
        pltpu.make_async_copy(shared_tab.at[sidx[b]], srow[b],
                              gsem[b]).wait()
        nxt = jnp.minimum(g + 1, NGROUP - 1)
        pltpu.async_copy(sidx_src(nxt), sidx[1 - b], isem[1 - b])
        pltpu.async_copy(vidx_src(nxt), vidx[1 - b], jsem[1 - b])
        pltpu.async_copy(srow[b], sout_dst(g), ssem[b])
      return carry

    lax.fori_loop(0, NGROUP // 2, body, 0)

    pltpu.make_async_copy(sidx_src(0), sidx[1], isem[1]).wait()
    pltpu.make_async_copy(vidx_src(0), vidx[1], jsem[1]).wait()
    for b in (0, 1):
      pltpu.make_async_copy(srow[b], sout_dst(0), ssem[b]).wait()
      pltpu.make_async_copy(vrow[b], vout_dst(0), vsem[b]).wait()

  return emb_kernel


_EMB_KERNEL = _build_kernel()


@jax.jit
def kernel(note, table):
  flat = note.reshape(-1)
  out = _EMB_KERNEL(flat, table)
  return out.reshape(BATCH, HIST, D)
